# baseline (speedup 1.0000x reference)
import jax
import jax.numpy as jnp
from jax.experimental import pallas as pl
from jax.experimental.pallas import tpu as pltpu

_BN = 512
_FP = 56


def _proj_body(xt_ref, fm_ref, dk_ref, w_ref, o_ref):
    i = pl.program_id(0)
    fm = fm_ref[i]
    dk = dk_ref[i]
    keep = ((fm != 0) & (dk != 1)).astype(jnp.float32)
    xt = xt_ref[:, pl.ds(i * _BN, _BN)]
    xm = xt * keep
    o_ref[...] = jax.lax.dot_general(
        xm, w_ref[...], (((0,), (0,)), ((), ())),
        preferred_element_type=jnp.float32)


def kernel(keypoints_xyc, feats_masks, drop_kps, W, b):
    B, S = feats_masks.shape
    N = B * S
    H, F = W.shape
    nblk = N // _BN
    feats = keypoints_xyc.reshape(N, F)
    xt = jnp.concatenate(
        [feats.T,
         jnp.ones((1, N), jnp.float32),
         jnp.zeros((_FP - F - 1, N), jnp.float32)], axis=0)
    fm = feats_masks.reshape(nblk, 1, _BN).astype(jnp.int8)
    dk = drop_kps.reshape(nblk, 1, _BN).astype(jnp.int8)
    waug = jnp.concatenate(
        [W.T, b.reshape(1, H), jnp.zeros((_FP - F - 1, H), jnp.float32)],
        axis=0)
    out = pl.pallas_call(
        _proj_body,
        grid=(nblk,),
        in_specs=[
            pl.BlockSpec((_FP, N), lambda i: (0, 0)),
            pl.BlockSpec((nblk, 1, _BN), lambda i: (0, 0, 0)),
            pl.BlockSpec((nblk, 1, _BN), lambda i: (0, 0, 0)),
            pl.BlockSpec((_FP, H), lambda i: (0, 0)),
        ],
        out_specs=pl.BlockSpec((_BN, H), lambda i: (i, 0)),
        out_shape=jax.ShapeDtypeStruct((N, H), jnp.float32),
        compiler_params=pltpu.CompilerParams(
            dimension_semantics=("parallel",),
        ),
    )(xt, fm, dk, waug)
    return out.reshape(B, S, H)


# speedup vs baseline: 1.0457x; 1.0457x over previous
import jax
import jax.numpy as jnp
from jax.experimental import pallas as pl
from jax.experimental.pallas import tpu as pltpu

_BN = 512
_FP = 56


def _proj_body(xt_ref, fm_ref, dk_ref, w_ref, o_ref):
    i = pl.program_id(0)
    fm = fm_ref[i]
    dk = dk_ref[i]
    keep = ((fm != 0) & (dk != 1)).astype(jnp.bfloat16)
    xt = xt_ref[:, pl.ds(i * _BN, _BN)]
    xm = xt * keep
    o_ref[...] = jax.lax.dot_general(
        xm, w_ref[...], (((0,), (0,)), ((), ())),
        preferred_element_type=jnp.float32)


def kernel(keypoints_xyc, feats_masks, drop_kps, W, b):
    B, S = feats_masks.shape
    N = B * S
    H, F = W.shape
    nblk = N // _BN
    feats = keypoints_xyc.reshape(N, F)
    xt = jnp.concatenate(
        [feats.T,
         jnp.ones((1, N), jnp.float32),
         jnp.zeros((_FP - F - 1, N), jnp.float32)], axis=0).astype(jnp.bfloat16)
    fm = feats_masks.reshape(nblk, 1, _BN).astype(jnp.int8)
    dk = drop_kps.reshape(nblk, 1, _BN).astype(jnp.int8)
    waug = jnp.concatenate(
        [W.T, b.reshape(1, H), jnp.zeros((_FP - F - 1, H), jnp.float32)],
        axis=0).astype(jnp.bfloat16)
    out = pl.pallas_call(
        _proj_body,
        grid=(nblk,),
        in_specs=[
            pl.BlockSpec((_FP, N), lambda i: (0, 0)),
            pl.BlockSpec((nblk, 1, _BN), lambda i: (0, 0, 0)),
            pl.BlockSpec((nblk, 1, _BN), lambda i: (0, 0, 0)),
            pl.BlockSpec((_FP, H), lambda i: (0, 0)),
        ],
        out_specs=pl.BlockSpec((_BN, H), lambda i: (i, 0)),
        out_shape=jax.ShapeDtypeStruct((N, H), jnp.float32),
        compiler_params=pltpu.CompilerParams(
            dimension_semantics=("parallel",),
        ),
    )(xt, fm, dk, waug)
    return out.reshape(B, S, H)


# speedup vs baseline: 1.3101x; 1.2528x over previous
import jax
import jax.numpy as jnp
from jax.experimental import pallas as pl
from jax.experimental.pallas import tpu as pltpu

_BN = 512
_FP = 56
_KO = 6


def _make_body(nblk, H):
    def _body(xt_ref, fm_ref, dk_ref, w_ref, o_hbm, obuf, o_sems):
        def o_copy(j, slot):
            return pltpu.make_async_copy(
                obuf.at[slot], o_hbm.at[pl.ds(j * _BN, _BN), :],
                o_sems.at[slot])

        def step(j, carry):
            slot = jax.lax.rem(j, _KO)

            @pl.when(j >= _KO)
            def _():
                o_copy(j - _KO, slot).wait()

            keep = ((fm_ref[j] != 0) & (dk_ref[j] != 1)).astype(jnp.bfloat16)
            xt = xt_ref[:, pl.ds(j * _BN, _BN)]
            xm = xt * keep
            obuf[slot] = jax.lax.dot_general(
                xm, w_ref[...], (((0,), (0,)), ((), ())),
                preferred_element_type=jnp.float32)
            o_copy(j, slot).start()
            return carry

        jax.lax.fori_loop(0, nblk, step, 0)

        for t in range(_KO):
            j = nblk - _KO + t
            o_copy(j, j % _KO).wait()

    return _body


def kernel(keypoints_xyc, feats_masks, drop_kps, W, b):
    B, S = feats_masks.shape
    N = B * S
    H, F = W.shape
    nblk = N // _BN
    feats = keypoints_xyc.reshape(N, F)
    xt = jnp.concatenate(
        [feats.T,
         jnp.ones((1, N), jnp.float32),
         jnp.zeros((_FP - F - 1, N), jnp.float32)],
        axis=0).astype(jnp.bfloat16)
    fm = feats_masks.reshape(nblk, 1, _BN).astype(jnp.int8)
    dk = drop_kps.reshape(nblk, 1, _BN).astype(jnp.int8)
    waug = jnp.concatenate(
        [W.T, b.reshape(1, H), jnp.zeros((_FP - F - 1, H), jnp.float32)],
        axis=0).astype(jnp.bfloat16)
    out = pl.pallas_call(
        _make_body(nblk, H),
        in_specs=[
            pl.BlockSpec(memory_space=pltpu.VMEM),
            pl.BlockSpec(memory_space=pltpu.VMEM),
            pl.BlockSpec(memory_space=pltpu.VMEM),
            pl.BlockSpec(memory_space=pltpu.VMEM),
        ],
        out_specs=pl.BlockSpec(memory_space=pl.ANY),
        out_shape=jax.ShapeDtypeStruct((N, H), jnp.float32),
        scratch_shapes=[
            pltpu.VMEM((_KO, _BN, H), jnp.float32),
            pltpu.SemaphoreType.DMA((_KO,)),
        ],
    )(xt, fm, dk, waug)
    return out.reshape(B, S, H)
